# baseline (speedup 1.0000x reference)
import functools

import jax
import jax.numpy as jnp
from jax import lax
from jax.experimental import pallas as pl
from jax.experimental.pallas import tpu as pltpu
from jax.experimental.pallas import tpu_sc as plsc

K = 64
B = 16384
NC = 2
NS = 16
NW = NC * NS
BPW = B // NW
CH = 128
L = 16
NCH = BPW // CH
NG = CH // L
SCALE = 1.0 / (K ** 0.5)


def _fpmc_body(uid_hbm, bp_hbm, iid_hbm, ui_hbm, iu_hbm, il_hbm, li_hbm,
               out_hbm, uid_v, bp_v, iid_v, a_v, b_v, out_v, sem):
    wid = lax.axis_index("s") * NC + lax.axis_index("c")
    base = wid * BPW

    pltpu.sync_copy(uid_hbm.at[pl.ds(base, BPW)], uid_v)
    pltpu.sync_copy(iid_hbm.at[pl.ds(base, BPW)], iid_v)
    pltpu.sync_copy(bp_hbm.at[pl.ds(base, BPW)], bp_v)

    lanes = lax.iota(jnp.int32, L)

    def run_phase(tab_a, idx_a, tab_b, idx_b, first):
        def chunk_body(c, _):
            off = c * CH
            cp_a = pltpu.async_copy(tab_a.at[idx_a.at[pl.ds(off, CH)]], a_v, sem)
            cp_b = pltpu.async_copy(tab_b.at[idx_b.at[pl.ds(off, CH)]], b_v, sem)
            cp_a.wait()
            cp_b.wait()

            def grp_body(g, _):
                r0 = g * L
                vec = jnp.zeros((L,), jnp.float32)
                for r in range(L):
                    acc = a_v[r0 + r, pl.ds(0, L)] * b_v[r0 + r, pl.ds(0, L)]
                    for j in range(1, K // L):
                        acc = acc + (a_v[r0 + r, pl.ds(j * L, L)]
                                     * b_v[r0 + r, pl.ds(j * L, L)])
                    for step in (8, 4, 2, 1):
                        acc = acc + acc.at[lanes ^ step].get(
                            mode="promise_in_bounds")
                    vec = jnp.where(lanes == r, acc, vec)
                dst = pl.ds(off + r0, L)
                if first:
                    out_v[dst] = vec
                else:
                    out_v[dst] = (out_v[dst] + vec) * SCALE
                return _

            return lax.fori_loop(0, NG, grp_body, None)

        lax.fori_loop(0, NCH, chunk_body, None)

    run_phase(ui_hbm, uid_v, iu_hbm, iid_v, first=True)
    run_phase(il_hbm, iid_v, li_hbm, bp_v, first=False)

    pltpu.sync_copy(out_v, out_hbm.at[pl.ds(base, BPW)])


_fpmc = functools.partial(
    pl.kernel,
    mesh=plsc.VectorSubcoreMesh(core_axis_name="c", subcore_axis_name="s"),
    compiler_params=pltpu.CompilerParams(use_tc_tiling_on_sc=False),
    out_type=jax.ShapeDtypeStruct((B,), jnp.float32),
    scratch_types=[
        pltpu.VMEM((BPW,), jnp.int32),
        pltpu.VMEM((BPW,), jnp.int32),
        pltpu.VMEM((BPW,), jnp.int32),
        pltpu.VMEM((CH, K), jnp.float32),
        pltpu.VMEM((CH, K), jnp.float32),
        pltpu.VMEM((BPW,), jnp.float32),
        pltpu.SemaphoreType.DMA,
    ],
)(_fpmc_body)


def kernel(uid, basket_prev, iid, UI, IU, IL, LI):
    return _fpmc(uid.astype(jnp.int32), basket_prev.astype(jnp.int32),
                 iid.astype(jnp.int32), UI, IU, IL, LI)


# speedup vs baseline: 1.0020x; 1.0020x over previous
import functools

import jax
import jax.numpy as jnp
from jax import lax
from jax.experimental import pallas as pl
from jax.experimental.pallas import tpu as pltpu
from jax.experimental.pallas import tpu_sc as plsc

K = 64
B = 16384
NC = 2
NS = 16
NW = NC * NS
BPW = B // NW
CH = 128
L = 16
NCH = BPW // CH
NG = CH // L
SCALE = 1.0 / (K ** 0.5)


def _pair_dots(idx_a_hbm, idx_b_hbm, tab_a, tab_b, base, ia_v, ib_v, a_v, b_v,
               out_v, sem, combine):
    pltpu.sync_copy(idx_a_hbm.at[pl.ds(base, BPW)], ia_v)
    pltpu.sync_copy(idx_b_hbm.at[pl.ds(base, BPW)], ib_v)

    lanes = lax.iota(jnp.int32, L)

    def chunk_body(c, _):
        off = c * CH
        cp_a = pltpu.async_copy(tab_a.at[ia_v.at[pl.ds(off, CH)]], a_v, sem)
        cp_b = pltpu.async_copy(tab_b.at[ib_v.at[pl.ds(off, CH)]], b_v, sem)
        cp_a.wait()
        cp_b.wait()

        def grp_body(g, _):
            r0 = g * L
            vec = jnp.zeros((L,), jnp.float32)
            for r in range(L):
                acc = a_v[r0 + r, pl.ds(0, L)] * b_v[r0 + r, pl.ds(0, L)]
                for j in range(1, K // L):
                    acc = acc + (a_v[r0 + r, pl.ds(j * L, L)]
                                 * b_v[r0 + r, pl.ds(j * L, L)])
                for step in (8, 4, 2, 1):
                    acc = acc + acc.at[lanes ^ step].get(
                        mode="promise_in_bounds")
                vec = jnp.where(lanes == r, acc, vec)
            combine(pl.ds(off + r0, L), vec)
            return _

        return lax.fori_loop(0, NG, grp_body, None)

    lax.fori_loop(0, NCH, chunk_body, None)


def _mf_body(uid_hbm, iid_hbm, ui_hbm, iu_hbm, out_hbm,
             ia_v, ib_v, a_v, b_v, out_v, sem):
    wid = lax.axis_index("s") * NC + lax.axis_index("c")
    base = wid * BPW

    def combine(dst, vec):
        out_v[dst] = vec

    _pair_dots(uid_hbm, iid_hbm, ui_hbm, iu_hbm, base, ia_v, ib_v, a_v, b_v,
               out_v, sem, combine)
    pltpu.sync_copy(out_v, out_hbm.at[pl.ds(base, BPW)])


def _fmc_body(iid_hbm, bp_hbm, il_hbm, li_hbm, mf_hbm, out_hbm,
              ia_v, ib_v, a_v, b_v, out_v, sem):
    wid = lax.axis_index("s") * NC + lax.axis_index("c")
    base = wid * BPW
    pltpu.sync_copy(mf_hbm.at[pl.ds(base, BPW)], out_v)

    def combine(dst, vec):
        out_v[dst] = (out_v[dst] + vec) * SCALE

    _pair_dots(iid_hbm, bp_hbm, il_hbm, li_hbm, base, ia_v, ib_v, a_v, b_v,
               out_v, sem, combine)
    pltpu.sync_copy(out_v, out_hbm.at[pl.ds(base, BPW)])


_SCRATCH = [
    pltpu.VMEM((BPW,), jnp.int32),
    pltpu.VMEM((BPW,), jnp.int32),
    pltpu.VMEM((CH, K), jnp.float32),
    pltpu.VMEM((CH, K), jnp.float32),
    pltpu.VMEM((BPW,), jnp.float32),
    pltpu.SemaphoreType.DMA,
]

_mf = functools.partial(
    pl.kernel,
    mesh=plsc.VectorSubcoreMesh(core_axis_name="c", subcore_axis_name="s"),
    compiler_params=pltpu.CompilerParams(use_tc_tiling_on_sc=False),
    out_type=jax.ShapeDtypeStruct((B,), jnp.float32),
    scratch_types=_SCRATCH,
)(_mf_body)

_fmc = functools.partial(
    pl.kernel,
    mesh=plsc.VectorSubcoreMesh(core_axis_name="c", subcore_axis_name="s"),
    compiler_params=pltpu.CompilerParams(use_tc_tiling_on_sc=False),
    out_type=jax.ShapeDtypeStruct((B,), jnp.float32),
    scratch_types=_SCRATCH,
)(_fmc_body)


def kernel(uid, basket_prev, iid, UI, IU, IL, LI):
    uid = uid.astype(jnp.int32)
    bp = basket_prev.astype(jnp.int32)
    iid = iid.astype(jnp.int32)
    mf = _mf(uid, iid, UI, IU)
    return _fmc(iid, bp, IL, LI, mf)
